# baseline (speedup 1.0000x reference)
import jax
import jax.numpy as jnp
from jax import lax
from jax.experimental import pallas as pl
from jax.experimental.pallas import tpu as pltpu
from jax.experimental.pallas import tpu_sc as plsc

NC = 2
NS = 16
NT = NC * NS
HN = 64
NP = 10240
CEDGE = 104
RING = 4
LOOK = 2
ZR = 64

_MESH = plsc.VectorSubcoreMesh(
    core_axis_name="c", subcore_axis_name="s", num_cores=NC, num_subcores=NS)
_SC_PARAMS = pltpu.CompilerParams(
    needs_layout_passes=False, use_tc_tiling_on_sc=False)


def _splat16(v):
    return jnp.full((16,), v, jnp.int32)


def _deg_partials(rid2, ew2):
    NCH = rid2.shape[0]
    CH = NCH // NS
    JR = CH // RING
    RPT = NP // NS

    def body(rid2_h, ew2_h, out_h, s_deg, ridl, ewl, zb, sems):
        c = lax.axis_index("c")
        s = lax.axis_index("s")
        r0 = s * RPT
        cbase = c * NP
        pltpu.sync_copy(rid2_h.at[pl.ds(s * CH, CH)], ridl)
        pltpu.sync_copy(ew2_h.at[pl.ds(s * CH, CH)], ewl)

        def zfill(i, carry):
            zb[pl.ds(i * 16, 16)] = jnp.zeros((16,), jnp.float32)
            return carry
        lax.fori_loop(0, RPT // 16, zfill, 0)
        pltpu.sync_copy(zb, s_deg.at[pl.ds(r0, RPT)])
        plsc.subcore_barrier()

        def jbody(j, carry):
            for i in range(RING):
                ch = j * RING + i

                @pl.when(j >= 1)
                def _():
                    pltpu.make_async_copy(
                        ewl.at[0], s_deg.at[ridl.at[0]], sems[i]).wait()
                pltpu.async_copy(
                    ewl.at[ch], s_deg.at[ridl.at[ch]], sems[i], add=True)
            return carry
        lax.fori_loop(0, JR, jbody, 0)
        for i in range(RING):
            pltpu.make_async_copy(
                ewl.at[0], s_deg.at[ridl.at[0]], sems[i]).wait()
        plsc.subcore_barrier()
        pltpu.sync_copy(s_deg.at[pl.ds(r0, RPT)],
                        out_h.at[pl.ds(cbase + r0, RPT)])

    return pl.kernel(
        body,
        out_type=jax.ShapeDtypeStruct((2 * NP,), jnp.float32),
        mesh=_MESH,
        compiler_params=_SC_PARAMS,
        scratch_types=[
            pltpu.VMEM_SHARED((NP,), jnp.float32),
            pltpu.VMEM((CH, CEDGE), jnp.int32),
            pltpu.VMEM((CH, CEDGE), jnp.float32),
            pltpu.VMEM((RPT,), jnp.float32),
            [pltpu.SemaphoreType.DMA for _ in range(RING)],
        ],
    )(rid2, ew2)


def _linear_and_dis(x, wt, b2, deg):
    N, D = x.shape
    BL = 1000

    def body(x_ref, wt_ref, b_ref, dp_ref, xo_ref, dis_ref):
        xo_ref[...] = (
            jnp.dot(x_ref[...], wt_ref[...], preferred_element_type=jnp.float32)
            + b_ref[...])
        deg_v = dp_ref[...]
        pos = deg_v > 0.0
        dis_ref[...] = jnp.where(
            pos, lax.rsqrt(jnp.where(pos, deg_v, 1.0)), 0.0)

    return pl.pallas_call(
        body,
        grid=(N // BL,),
        in_specs=[
            pl.BlockSpec((BL, D), lambda i: (i, 0)),
            pl.BlockSpec((D, D), lambda i: (0, 0)),
            pl.BlockSpec((1, D), lambda i: (0, 0)),
            pl.BlockSpec((BL, 1), lambda i: (i, 0)),
        ],
        out_specs=[
            pl.BlockSpec((BL, D), lambda i: (i, 0)),
            pl.BlockSpec((BL, 1), lambda i: (i, 0)),
        ],
        out_shape=[
            jax.ShapeDtypeStruct((N, D), jnp.float32),
            jax.ShapeDtypeStruct((N, 1), jnp.float32),
        ],
    )(x, wt, b2, deg)


def _norm_weights(row, col, ew, dis, N):
    E2 = row.shape[0]
    EW = E2 // NT

    def body(row_h, col_h, ew_h, dis_h, out_h, disb, rb, cb, eb, ob):
        base = (lax.axis_index("c") * NS + lax.axis_index("s")) * EW
        pltpu.sync_copy(dis_h, disb)
        pltpu.sync_copy(row_h.at[pl.ds(base, EW)], rb)
        pltpu.sync_copy(col_h.at[pl.ds(base, EW)], cb)
        pltpu.sync_copy(ew_h.at[pl.ds(base, EW)], eb)

        def g16(g, carry):
            sl = pl.ds(g * 16, 16)
            dr = plsc.load_gather(disb, [rb[sl]])
            dc = plsc.load_gather(disb, [cb[sl]])
            ob[sl] = dr * eb[sl] * dc
            return carry
        lax.fori_loop(0, EW // 16, g16, 0)
        pltpu.sync_copy(ob, out_h.at[pl.ds(base, EW)])

    return pl.kernel(
        body,
        out_type=jax.ShapeDtypeStruct((E2,), jnp.float32),
        mesh=_MESH,
        compiler_params=_SC_PARAMS,
        scratch_types=[
            pltpu.VMEM((N,), jnp.float32),
            pltpu.VMEM((EW,), jnp.int32),
            pltpu.VMEM((EW,), jnp.int32),
            pltpu.VMEM((EW,), jnp.float32),
            pltpu.VMEM((EW,), jnp.float32),
        ],
    )(row, col, ew, dis)


def _propagate(xs0, echunk, K):
    NCH = echunk.shape[1]
    CH = NCH // NS
    JR = CH // RING
    RPT = NP // NS

    def body(xs0_h, ec_h, *rest):
        xk_h = rest[:K]
        s_acc, ebuf, rows, zst, gsem, ssem = rest[K:]
        c = lax.axis_index("c")
        s = lax.axis_index("s")
        r0 = s * RPT
        cbase = c * NP

        def zfill(i, carry):
            for g in range(HN // 16):
                zst[i, pl.ds(g * 16, 16)] = jnp.zeros((16,), jnp.float32)
            return carry
        lax.fori_loop(0, ZR, zfill, 0)

        def zacc(sc, carry):
            pltpu.sync_copy(zst, s_acc.at[pl.ds(r0 + sc * ZR, ZR)])
            return carry
        lax.fori_loop(0, RPT // ZR, zacc, 0)
        plsc.subcore_barrier()

        def efetch(q, bf):
            pltpu.sync_copy(ec_h.at[c, s * CH + q], ebuf[bf])

        def gstart(xsrc, q, bf):
            pltpu.async_copy(xsrc.at[ebuf[bf].at[0]], rows[bf], gsem[bf])

        def gwait(xsrc, bf):
            pltpu.make_async_copy(
                xsrc.at[ebuf[bf].at[0]], rows[bf], gsem[bf]).wait()

        def sstart(bf):
            pltpu.async_copy(
                rows[bf], s_acc.at[ebuf[bf].at[1]], ssem[bf], add=True)

        def swait(bf):
            pltpu.make_async_copy(
                rows[bf], s_acc.at[ebuf[bf].at[1]], ssem[bf]).wait()

        def step(xsrc, xdst):
            for bq in range(LOOK):
                efetch(bq, bq)
                gstart(xsrc, bq, bq)

            def jbody(j, carry):
                for i in range(RING):
                    q = j * RING + i
                    gwait(xsrc, i)

                    def ebody(eg, ecarry):
                        e0 = eg * 4
                        for u in range(4):
                            wi = plsc.load_gather(
                                ebuf[i], [_splat16(2), _splat16(e0 + u)])
                            wv = plsc.bitcast(wi, jnp.float32)
                            for g in range(HN // 16):
                                sl = pl.ds(g * 16, 16)
                                rows[i][e0 + u, sl] = rows[i][e0 + u, sl] * wv
                        return ecarry
                    lax.fori_loop(0, CEDGE // 4, ebody, 0)
                    sstart(i)
                    bn = (i + LOOK) % RING

                    @pl.when(q + LOOK < CH)
                    def _():
                        @pl.when(q >= RING - LOOK)
                        def _():
                            swait(bn)
                        efetch(q + LOOK, bn)
                        gstart(xsrc, q + LOOK, bn)
                return carry
            lax.fori_loop(0, JR, jbody, 0)
            for bq in range(RING):
                swait(bq)

            plsc.subcore_barrier()
            pltpu.sync_copy(s_acc.at[pl.ds(r0, RPT)],
                            xdst.at[pl.ds(cbase + r0, RPT)])

            def zacc2(sc, carry):
                pltpu.sync_copy(zst, s_acc.at[pl.ds(r0 + sc * ZR, ZR)])
                return carry
            lax.fori_loop(0, RPT // ZR, zacc2, 0)
            plsc.subcore_barrier()

        srcs = [xs0_h] + list(xk_h[:-1])
        for k in range(K):
            step(srcs[k], xk_h[k])

    out = pl.kernel(
        body,
        out_type=[jax.ShapeDtypeStruct((2 * NP, HN), jnp.float32)
                  for _ in range(K)],
        mesh=_MESH,
        compiler_params=_SC_PARAMS,
        scratch_types=[
            pltpu.VMEM_SHARED((NP, HN), jnp.float32),
            [pltpu.VMEM((3, CEDGE), jnp.int32) for _ in range(RING)],
            [pltpu.VMEM((CEDGE, HN), jnp.float32) for _ in range(RING)],
            pltpu.VMEM((ZR, HN), jnp.float32),
            [pltpu.SemaphoreType.DMA for _ in range(RING)],
            [pltpu.SemaphoreType.DMA for _ in range(RING)],
        ],
    )(xs0, echunk)
    return out


def _combine(xs, tmp16):
    M = xs[0].shape[0]
    BL = 1024
    KP1 = len(xs)

    def body(*refs):
        t_ref = refs[KP1]
        o_ref = refs[KP1 + 1]
        acc = refs[0][...] * t_ref[0, 0]
        for k in range(1, KP1):
            acc = acc + refs[k][...] * t_ref[0, k]
        o_ref[...] = acc

    return pl.pallas_call(
        body,
        grid=(M // BL,),
        in_specs=[pl.BlockSpec((BL, HN), lambda i: (i, 0))
                  for _ in range(KP1)]
        + [pl.BlockSpec((1, 16), lambda i: (0, 0))],
        out_specs=pl.BlockSpec((BL, HN), lambda i: (i, 0)),
        out_shape=jax.ShapeDtypeStruct((M, HN), jnp.float32),
    )(*xs, tmp16)


def kernel(x, edge_index, edge_weight, W, b, temp):
    N, D = x.shape
    E = edge_index.shape[1]
    K = temp.shape[0] - 1
    row = edge_index[0]
    col = edge_index[1]

    NCH = E // 100
    rid2 = jnp.zeros((NCH, CEDGE), jnp.int32).at[:, :100].set(
        row.reshape(NCH, 100))
    cid2 = jnp.zeros((NCH, CEDGE), jnp.int32).at[:, :100].set(
        col.reshape(NCH, 100))
    ew2 = jnp.zeros((NCH, CEDGE), jnp.float32).at[:, :100].set(
        edge_weight.reshape(NCH, 100))

    deg = _deg_partials(rid2, ew2)
    xp, dis = _linear_and_dis(x, W.T, b.reshape(1, D), deg[:N].reshape(N, 1))
    wn = _norm_weights(rid2.reshape(-1), cid2.reshape(-1),
                       ew2.reshape(-1), dis.reshape(N), N)
    wn2i = lax.bitcast_convert_type(wn.reshape(NCH, CEDGE), jnp.int32)

    echunk = jnp.stack([
        jnp.stack([cid2, rid2, wn2i], axis=1),
        jnp.stack([cid2 + NP, rid2, wn2i], axis=1),
    ])

    xs0 = (jnp.zeros((2 * NP, HN), jnp.float32)
           .at[:N].set(xp[:, :HN])
           .at[NP:NP + N].set(xp[:, HN:]))
    tmp16 = jnp.zeros((1, 16), jnp.float32).at[0, : K + 1].set(temp)

    xks = _propagate(xs0, echunk, K)
    hid = _combine([xs0] + list(xks), tmp16)
    return jnp.concatenate([hid[:N], hid[NP:NP + N]], axis=1)


# speedup vs baseline: 1.0065x; 1.0065x over previous
import jax
import jax.numpy as jnp
from jax import lax
from jax.experimental import pallas as pl
from jax.experimental.pallas import tpu as pltpu
from jax.experimental.pallas import tpu_sc as plsc

NC = 2
NS = 16
NT = NC * NS
HN = 64
NP = 10240
CEDGE = 104
RING = 4
LOOK = 2
EDEEP = 8
ELEAD = 6
ZR = 64

_MESH = plsc.VectorSubcoreMesh(
    core_axis_name="c", subcore_axis_name="s", num_cores=NC, num_subcores=NS)
_SC_PARAMS = pltpu.CompilerParams(
    needs_layout_passes=False, use_tc_tiling_on_sc=False)


def _splat16(v):
    return jnp.full((16,), v, jnp.int32)


def _deg_partials(rid2, ew2):
    NCH = rid2.shape[0]
    CH = NCH // NS
    JR = CH // RING
    RPT = NP // NS

    def body(rid2_h, ew2_h, out_h, s_deg, ridl, ewl, zb, sems):
        c = lax.axis_index("c")
        s = lax.axis_index("s")
        r0 = s * RPT
        cbase = c * NP
        pltpu.sync_copy(rid2_h.at[pl.ds(s * CH, CH)], ridl)
        pltpu.sync_copy(ew2_h.at[pl.ds(s * CH, CH)], ewl)

        def zfill(i, carry):
            zb[pl.ds(i * 16, 16)] = jnp.zeros((16,), jnp.float32)
            return carry
        lax.fori_loop(0, RPT // 16, zfill, 0)
        pltpu.sync_copy(zb, s_deg.at[pl.ds(r0, RPT)])
        plsc.subcore_barrier()

        def jbody(j, carry):
            for i in range(RING):
                ch = j * RING + i

                @pl.when(j >= 1)
                def _():
                    pltpu.make_async_copy(
                        ewl.at[0], s_deg.at[ridl.at[0]], sems[i]).wait()
                pltpu.async_copy(
                    ewl.at[ch], s_deg.at[ridl.at[ch]], sems[i], add=True)
            return carry
        lax.fori_loop(0, JR, jbody, 0)
        for i in range(RING):
            pltpu.make_async_copy(
                ewl.at[0], s_deg.at[ridl.at[0]], sems[i]).wait()
        plsc.subcore_barrier()
        pltpu.sync_copy(s_deg.at[pl.ds(r0, RPT)],
                        out_h.at[pl.ds(cbase + r0, RPT)])

    return pl.kernel(
        body,
        out_type=jax.ShapeDtypeStruct((2 * NP,), jnp.float32),
        mesh=_MESH,
        compiler_params=_SC_PARAMS,
        scratch_types=[
            pltpu.VMEM_SHARED((NP,), jnp.float32),
            pltpu.VMEM((CH, CEDGE), jnp.int32),
            pltpu.VMEM((CH, CEDGE), jnp.float32),
            pltpu.VMEM((RPT,), jnp.float32),
            [pltpu.SemaphoreType.DMA for _ in range(RING)],
        ],
    )(rid2, ew2)


def _linear_and_dis(x, wt, b2, deg):
    N, D = x.shape
    BL = 1000

    def body(x_ref, wt_ref, b_ref, dp_ref, xo_ref, dis_ref):
        xo_ref[...] = (
            jnp.dot(x_ref[...], wt_ref[...], preferred_element_type=jnp.float32)
            + b_ref[...])
        deg_v = dp_ref[...]
        pos = deg_v > 0.0
        dis_ref[...] = jnp.where(
            pos, lax.rsqrt(jnp.where(pos, deg_v, 1.0)), 0.0)

    return pl.pallas_call(
        body,
        grid=(N // BL,),
        in_specs=[
            pl.BlockSpec((BL, D), lambda i: (i, 0)),
            pl.BlockSpec((D, D), lambda i: (0, 0)),
            pl.BlockSpec((1, D), lambda i: (0, 0)),
            pl.BlockSpec((BL, 1), lambda i: (i, 0)),
        ],
        out_specs=[
            pl.BlockSpec((BL, D), lambda i: (i, 0)),
            pl.BlockSpec((BL, 1), lambda i: (i, 0)),
        ],
        out_shape=[
            jax.ShapeDtypeStruct((N, D), jnp.float32),
            jax.ShapeDtypeStruct((N, 1), jnp.float32),
        ],
    )(x, wt, b2, deg)


def _norm_weights(row, col, ew, dis, N):
    E2 = row.shape[0]
    EW = E2 // NT

    def body(row_h, col_h, ew_h, dis_h, out_h, disb, rb, cb, eb, ob):
        base = (lax.axis_index("c") * NS + lax.axis_index("s")) * EW
        pltpu.sync_copy(dis_h, disb)
        pltpu.sync_copy(row_h.at[pl.ds(base, EW)], rb)
        pltpu.sync_copy(col_h.at[pl.ds(base, EW)], cb)
        pltpu.sync_copy(ew_h.at[pl.ds(base, EW)], eb)

        def g16(g, carry):
            sl = pl.ds(g * 16, 16)
            dr = plsc.load_gather(disb, [rb[sl]])
            dc = plsc.load_gather(disb, [cb[sl]])
            ob[sl] = dr * eb[sl] * dc
            return carry
        lax.fori_loop(0, EW // 16, g16, 0)
        pltpu.sync_copy(ob, out_h.at[pl.ds(base, EW)])

    return pl.kernel(
        body,
        out_type=jax.ShapeDtypeStruct((E2,), jnp.float32),
        mesh=_MESH,
        compiler_params=_SC_PARAMS,
        scratch_types=[
            pltpu.VMEM((N,), jnp.float32),
            pltpu.VMEM((EW,), jnp.int32),
            pltpu.VMEM((EW,), jnp.int32),
            pltpu.VMEM((EW,), jnp.float32),
            pltpu.VMEM((EW,), jnp.float32),
        ],
    )(row, col, ew, dis)


def _propagate(xs0, echunk, K):
    NCH = echunk.shape[1]
    CH = NCH // NS
    JR = CH // RING
    RPT = NP // NS

    def body(xs0_h, ec_h, *rest):
        xk_h = rest[:K]
        s_acc, ebuf, rows, zst, gsem, ssem, esem = rest[K:]
        c = lax.axis_index("c")
        s = lax.axis_index("s")
        r0 = s * RPT
        cbase = c * NP

        def zfill(i, carry):
            for g in range(HN // 16):
                zst[i, pl.ds(g * 16, 16)] = jnp.zeros((16,), jnp.float32)
            return carry
        lax.fori_loop(0, ZR, zfill, 0)

        def zacc(sc, carry):
            pltpu.sync_copy(zst, s_acc.at[pl.ds(r0 + sc * ZR, ZR)])
            return carry
        lax.fori_loop(0, RPT // ZR, zacc, 0)
        plsc.subcore_barrier()

        def estart(q, es):
            pltpu.async_copy(ec_h.at[c, s * CH + q], ebuf[es], esem[es])

        def ewait(es):
            pltpu.make_async_copy(
                ec_h.at[c, 0], ebuf[es], esem[es]).wait()

        def gstart(xsrc, bf, es):
            pltpu.async_copy(xsrc.at[ebuf[es].at[0]], rows[bf], gsem[bf])

        def gwait(xsrc, bf, es):
            pltpu.make_async_copy(
                xsrc.at[ebuf[es].at[0]], rows[bf], gsem[bf]).wait()

        def sstart(bf, es):
            pltpu.async_copy(
                rows[bf], s_acc.at[ebuf[es].at[1]], ssem[bf], add=True)

        def swait(bf, es):
            pltpu.make_async_copy(
                rows[bf], s_acc.at[ebuf[es].at[1]], ssem[bf]).wait()

        def step(xsrc, xdst):
            for bq in range(ELEAD):
                estart(bq, bq)
            for bq in range(LOOK):
                ewait(bq)
                gstart(xsrc, bq, bq)

            def jbody(j, carry):
                for jj in range(2):
                    for i in range(RING):
                        q = (2 * j + jj) * RING + i
                        es = (jj * RING + i) % EDEEP
                        gwait(xsrc, i, es)

                        def ebody(e, ecarry):
                            wi = plsc.load_gather(
                                ebuf[es], [_splat16(2), _splat16(e)])
                            wv = plsc.bitcast(wi, jnp.float32)
                            for g in range(HN // 16):
                                sl = pl.ds(g * 16, 16)
                                rows[i][e, sl] = rows[i][e, sl] * wv
                            return ecarry
                        lax.fori_loop(0, CEDGE, ebody, 0)
                        sstart(i, es)
                        bn = (i + LOOK) % RING
                        en = (es + LOOK) % EDEEP

                        @pl.when(q + LOOK < CH)
                        def _():
                            @pl.when(q >= RING - LOOK)
                            def _():
                                swait(bn, (es - LOOK) % EDEEP)

                            @pl.when(q + ELEAD < CH)
                            def _():
                                estart(q + ELEAD, (es + ELEAD) % EDEEP)
                            ewait(en)
                            gstart(xsrc, bn, en)
                return carry
            lax.fori_loop(0, JR // 2, jbody, 0)
            for bq in range(RING):
                es_last = (CH - RING + bq) % EDEEP
                swait(bq, es_last)

            plsc.subcore_barrier()
            pltpu.sync_copy(s_acc.at[pl.ds(r0, RPT)],
                            xdst.at[pl.ds(cbase + r0, RPT)])

            def zacc2(sc, carry):
                pltpu.sync_copy(zst, s_acc.at[pl.ds(r0 + sc * ZR, ZR)])
                return carry
            lax.fori_loop(0, RPT // ZR, zacc2, 0)
            plsc.subcore_barrier()

        srcs = [xs0_h] + list(xk_h[:-1])
        for k in range(K):
            step(srcs[k], xk_h[k])

    out = pl.kernel(
        body,
        out_type=[jax.ShapeDtypeStruct((2 * NP, HN), jnp.float32)
                  for _ in range(K)],
        mesh=_MESH,
        compiler_params=_SC_PARAMS,
        scratch_types=[
            pltpu.VMEM_SHARED((NP, HN), jnp.float32),
            [pltpu.VMEM((3, CEDGE), jnp.int32) for _ in range(EDEEP)],
            [pltpu.VMEM((CEDGE, HN), jnp.float32) for _ in range(RING)],
            pltpu.VMEM((ZR, HN), jnp.float32),
            [pltpu.SemaphoreType.DMA for _ in range(RING)],
            [pltpu.SemaphoreType.DMA for _ in range(RING)],
            [pltpu.SemaphoreType.DMA for _ in range(EDEEP)],
        ],
    )(xs0, echunk)
    return out


def _combine(xs, tmp16):
    M = xs[0].shape[0]
    BL = 1024
    KP1 = len(xs)

    def body(*refs):
        t_ref = refs[KP1]
        o_ref = refs[KP1 + 1]
        acc = refs[0][...] * t_ref[0, 0]
        for k in range(1, KP1):
            acc = acc + refs[k][...] * t_ref[0, k]
        o_ref[...] = acc

    return pl.pallas_call(
        body,
        grid=(M // BL,),
        in_specs=[pl.BlockSpec((BL, HN), lambda i: (i, 0))
                  for _ in range(KP1)]
        + [pl.BlockSpec((1, 16), lambda i: (0, 0))],
        out_specs=pl.BlockSpec((BL, HN), lambda i: (i, 0)),
        out_shape=jax.ShapeDtypeStruct((M, HN), jnp.float32),
    )(*xs, tmp16)


def kernel(x, edge_index, edge_weight, W, b, temp):
    N, D = x.shape
    E = edge_index.shape[1]
    K = temp.shape[0] - 1
    row = edge_index[0]
    col = edge_index[1]

    NCH = E // 100
    rid2 = jnp.zeros((NCH, CEDGE), jnp.int32).at[:, :100].set(
        row.reshape(NCH, 100))
    cid2 = jnp.zeros((NCH, CEDGE), jnp.int32).at[:, :100].set(
        col.reshape(NCH, 100))
    ew2 = jnp.zeros((NCH, CEDGE), jnp.float32).at[:, :100].set(
        edge_weight.reshape(NCH, 100))

    deg = _deg_partials(rid2, ew2)
    xp, dis = _linear_and_dis(x, W.T, b.reshape(1, D), deg[:N].reshape(N, 1))
    wn = _norm_weights(rid2.reshape(-1), cid2.reshape(-1),
                       ew2.reshape(-1), dis.reshape(N), N)
    wn2i = lax.bitcast_convert_type(wn.reshape(NCH, CEDGE), jnp.int32)

    echunk = jnp.stack([
        jnp.stack([cid2, rid2, wn2i], axis=1),
        jnp.stack([cid2 + NP, rid2, wn2i], axis=1),
    ])

    xs0 = (jnp.zeros((2 * NP, HN), jnp.float32)
           .at[:N].set(xp[:, :HN])
           .at[NP:NP + N].set(xp[:, HN:]))
    tmp16 = jnp.zeros((1, 16), jnp.float32).at[0, : K + 1].set(temp)

    xks = _propagate(xs0, echunk, K)
    hid = _combine([xs0] + list(xks), tmp16)
    return jnp.concatenate([hid[:N], hid[NP:NP + N]], axis=1)
